# baseline (speedup 1.0000x reference)
import jax
import jax.numpy as jnp
from jax import lax
from jax.experimental import pallas as pl
from jax.experimental.pallas import tpu as pltpu
from jax.experimental.pallas import tpu_sc as plsc

NUM_CLASSES = 100000
HD = 128
N = 16384

NC = 2
NS = 16

BLOCK = 10240
BLOCKS_PER_CORE = 5
DUMMY = BLOCK
ACC_ROWS = BLOCK + 8
LPT = N // NS
LIST_CAP = LPT + 128
CORE_ROWS = BLOCK * BLOCKS_PER_CORE
AROWS = 800
ACHUNKS = CORE_ROWS // NS // AROWS


def _body(labels_hbm, hv_hbm, w_hbm, out_hbm,
          labels_v, zbuf, stage, src_list, dst_list,
          srcidx, dstidx, lidx, gidx, acc, sema, semg):
    c = lax.axis_index("c")
    s = lax.axis_index("s")
    lab_base = s * LPT
    iota16 = lax.iota(jnp.int32, 16)
    dummy16 = jnp.full((16,), DUMMY, jnp.int32)
    zero16 = jnp.zeros((16,), jnp.float32)

    a_base = c * CORE_ROWS + s * (AROWS * ACHUNKS)
    for q in range(ACHUNKS):
        a_start = a_base + q * AROWS

        @pl.when(a_start < NUM_CLASSES)
        def _():
            pltpu.async_copy(
                w_hbm.at[pl.ds(a_start, AROWS)],
                out_hbm.at[pl.ds(a_start, AROWS)],
                sema,
            )

    pltpu.sync_copy(labels_hbm.at[pl.ds(lab_base, LPT)], labels_v)

    def _zrow(i, carry):
        for g in range(HD // 16):
            zbuf[i, pl.ds(g * 16, 16)] = zero16
        return carry

    lax.fori_loop(0, 64, _zrow, 0)

    for b in range(BLOCKS_PER_CORE):
        blo = (c * BLOCKS_PER_CORE + b) * BLOCK
        bhi = blo + BLOCK

        zoff = s * (BLOCK // NS)
        for z in range(BLOCK // NS // 64):
            pltpu.sync_copy(zbuf, acc.at[pl.ds(zoff + z * 64, 64)])

        def _prefill(i, carry):
            src_list[pl.ds(i * 16, 16)] = dummy16
            dst_list[pl.ds(i * 16, 16)] = dummy16
            return carry

        lax.fori_loop(0, LPT // 16, _prefill, 0)

        def _scan(j, cnt):
            lab = labels_v[pl.ds(j * 16, 16)]
            m = (lab >= blo) & (lab < bhi)
            plsc.store_compressed(dst_list.at[pl.ds(cnt, 16)], lab - blo, mask=m)
            plsc.store_compressed(
                src_list.at[pl.ds(cnt, 16)], lab_base + j * 16 + iota16, mask=m
            )
            return cnt + jnp.sum(m.astype(jnp.int32))

        cnt = lax.fori_loop(0, LPT // 16, _scan, 0)
        nch = (cnt + 127) >> 7

        plsc.subcore_barrier()

        def _bchunk(k, carry):
            base = k * 128
            for g in range(8):
                dstidx[pl.ds(g * 16, 16)] = dst_list[pl.ds(base + g * 16, 16)]
            pltpu.async_copy(
                hv_hbm.at[src_list.at[pl.ds(base, 128)]], stage, semg
            ).wait()
            pltpu.sync_copy(stage, acc.at[dstidx], add=True)
            return carry

        lax.fori_loop(0, nch, _bchunk, 0)

        if b == 0:
            for q in range(ACHUNKS):
                a_start = a_base + q * AROWS

                @pl.when(a_start < NUM_CLASSES)
                def _():
                    pltpu.make_async_copy(
                        w_hbm.at[pl.ds(a_start, AROWS)],
                        out_hbm.at[pl.ds(a_start, AROWS)],
                        sema,
                    ).wait()

        plsc.subcore_barrier()

        def _cchunk(k, carry):
            base = k * 128
            g0 = dst_list[pl.ds(base, 16)]
            first = jnp.broadcast_to(jnp.min(g0), (16,))
            for g in range(8):
                v = dst_list[pl.ds(base + g * 16, 16)]
                v = jnp.where(v == DUMMY, first, v)
                lidx[pl.ds(g * 16, 16)] = v
                gidx[pl.ds(g * 16, 16)] = v + blo
            pltpu.async_copy(w_hbm.at[gidx], stage, semg).wait()
            pltpu.async_copy(acc.at[lidx], stage, semg, add=True).wait()
            pltpu.sync_copy(stage, out_hbm.at[gidx])
            return carry

        lax.fori_loop(0, nch, _cchunk, 0)

        plsc.subcore_barrier()


@jax.jit
def _scatter_add(labels, hv, classify_weights):
    mesh = plsc.VectorSubcoreMesh(
        core_axis_name="c", subcore_axis_name="s", num_cores=NC, num_subcores=NS
    )
    return pl.kernel(
        _body,
        out_type=jax.ShapeDtypeStruct((NUM_CLASSES, HD), jnp.float32),
        mesh=mesh,
        compiler_params=pltpu.CompilerParams(needs_layout_passes=False),
        scratch_types=[
            pltpu.VMEM((LPT,), jnp.int32),
            pltpu.VMEM((64, HD), jnp.float32),
            pltpu.VMEM((128, HD), jnp.float32),
            pltpu.VMEM((LIST_CAP,), jnp.int32),
            pltpu.VMEM((LIST_CAP,), jnp.int32),
            pltpu.VMEM((128,), jnp.int32),
            pltpu.VMEM((128,), jnp.int32),
            pltpu.VMEM((128,), jnp.int32),
            pltpu.VMEM((128,), jnp.int32),
            pltpu.VMEM_SHARED((ACC_ROWS, HD), jnp.float32),
            pltpu.SemaphoreType.DMA,
            pltpu.SemaphoreType.DMA,
        ],
    )(labels, hv, classify_weights)


def kernel(labels, hv, classify_weights):
    return _scatter_add(labels, hv, classify_weights)


# speedup vs baseline: 7.0544x; 7.0544x over previous
import jax
import jax.numpy as jnp
from jax import lax
from jax.experimental import pallas as pl
from jax.experimental.pallas import tpu as pltpu
from jax.experimental.pallas import tpu_sc as plsc

NUM_CLASSES = 100000
HD = 128
N = 16384

NC = 2
NS = 16

BLOCK = 12800
NB = 4
DUMMY = BLOCK
ACC_ROWS = BLOCK + 8
LPT = N // NS
LIST_CAP = LPT + 128
RPT = BLOCK // NS
CH = 64
MAXCH = LPT // CH


def _body(labels_hbm, hv_hbm, w_hbm, out_hbm,
          labels_v, stage, src_list, dst_list, dstidx, acc,
          sem_i, sem_o, semg0, semg1):
    c = lax.axis_index("c")
    s = lax.axis_index("s")
    lab_base = s * LPT
    iota16 = lax.iota(jnp.int32, 16)
    dummy16 = jnp.full((16,), DUMMY, jnp.int32)

    pltpu.sync_copy(labels_hbm.at[pl.ds(lab_base, LPT)], labels_v)

    def blo_of(b):
        return (c * NB + b) * BLOCK

    def fire_init(b):
        row0 = blo_of(b) + s * RPT

        @pl.when(row0 < NUM_CLASSES)
        def _():
            pltpu.async_copy(
                w_hbm.at[pl.ds(row0, RPT)], acc.at[pl.ds(s * RPT, RPT)], sem_i
            )

    def wait_init(b):
        row0 = blo_of(b) + s * RPT

        @pl.when(row0 < NUM_CLASSES)
        def _():
            pltpu.make_async_copy(
                w_hbm.at[pl.ds(row0, RPT)], acc.at[pl.ds(s * RPT, RPT)], sem_i
            ).wait()

    def do_scan(b):
        blo = blo_of(b)
        bhi = blo + BLOCK

        def _prefill(i, carry):
            src_list[pl.ds(i * 16, 16)] = dummy16
            dst_list[pl.ds(i * 16, 16)] = dummy16
            return carry

        lax.fori_loop(0, LPT // 16, _prefill, 0)

        def _scan(j, cnt):
            lab = labels_v[pl.ds(j * 16, 16)]
            m = (lab >= blo) & (lab < bhi)
            plsc.store_compressed(dst_list.at[pl.ds(cnt, 16)], lab - blo, mask=m)
            plsc.store_compressed(
                src_list.at[pl.ds(cnt, 16)], lab_base + j * 16 + iota16, mask=m
            )
            return cnt + jnp.sum(m.astype(jnp.int32))

        cnt = lax.fori_loop(0, LPT // 16, _scan, 0)
        return (cnt + CH - 1) >> 6

    def fire_gather(k, par, sem):
        pltpu.async_copy(
            hv_hbm.at[src_list.at[pl.ds(k * CH, CH)]], stage.at[par], sem
        )

    def wait_gather(k, par, sem):
        pltpu.make_async_copy(
            hv_hbm.at[src_list.at[pl.ds(k * CH, CH)]], stage.at[par], sem
        ).wait()

    fire_init(0)
    nch = do_scan(0)

    for b in range(NB):
        wait_init(b)
        plsc.subcore_barrier()

        @pl.when(0 < nch)
        def _():
            fire_gather(0, 0, semg0)

        for k in range(MAXCH):
            par = k % 2
            sem = semg0 if par == 0 else semg1
            npar = (k + 1) % 2
            nsem = semg0 if npar == 0 else semg1

            @pl.when(k + 1 < nch)
            def _():
                fire_gather(k + 1, npar, nsem)

            @pl.when(k < nch)
            def _():
                base = k * CH
                for g in range(CH // 16):
                    dstidx[pl.ds(g * 16, 16)] = dst_list[pl.ds(base + g * 16, 16)]
                wait_gather(k, par, sem)
                pltpu.sync_copy(stage.at[par], acc.at[dstidx], add=True)

        plsc.subcore_barrier()

        row0 = blo_of(b) + s * RPT

        @pl.when(row0 < NUM_CLASSES)
        def _():
            pltpu.async_copy(
                acc.at[pl.ds(s * RPT, RPT)], out_hbm.at[pl.ds(row0, RPT)], sem_o
            )

        if b + 1 < NB:
            nch = do_scan(b + 1)

        @pl.when(row0 < NUM_CLASSES)
        def _():
            pltpu.make_async_copy(
                acc.at[pl.ds(s * RPT, RPT)], out_hbm.at[pl.ds(row0, RPT)], sem_o
            ).wait()

        if b + 1 < NB:
            fire_init(b + 1)


@jax.jit
def _scatter_add(labels, hv, classify_weights):
    mesh = plsc.VectorSubcoreMesh(
        core_axis_name="c", subcore_axis_name="s", num_cores=NC, num_subcores=NS
    )
    return pl.kernel(
        _body,
        out_type=jax.ShapeDtypeStruct((NUM_CLASSES, HD), jnp.float32),
        mesh=mesh,
        compiler_params=pltpu.CompilerParams(needs_layout_passes=False),
        scratch_types=[
            pltpu.VMEM((LPT,), jnp.int32),
            pltpu.VMEM((2, CH, HD), jnp.float32),
            pltpu.VMEM((LIST_CAP,), jnp.int32),
            pltpu.VMEM((LIST_CAP,), jnp.int32),
            pltpu.VMEM((CH,), jnp.int32),
            pltpu.VMEM_SHARED((ACC_ROWS, HD), jnp.float32),
            pltpu.SemaphoreType.DMA,
            pltpu.SemaphoreType.DMA,
            pltpu.SemaphoreType.DMA,
            pltpu.SemaphoreType.DMA,
        ],
    )(labels, hv, classify_weights)


def kernel(labels, hv, classify_weights):
    return _scatter_add(labels, hv, classify_weights)
